# baseline (speedup 1.0000x reference)
import jax
import jax.numpy as jnp
from jax.experimental import pallas as pl
from jax.experimental.pallas import tpu as pltpu

_CHUNK = 200
_NBUF = 4


def _start_copy(adj_hbm, bufs, sems, idx, slot):
    pltpu.make_async_copy(
        adj_hbm.at[pl.ds(idx * _CHUNK, _CHUNK), :],
        bufs.at[slot],
        sems.at[slot],
    ).start()


def _fused_body(adj_hbm, x_hbm, w_ref, o_ref,
                sup_ref, bufs, x_ref, x_sem, sems):
    c = pl.program_id(0)
    nc = pl.num_programs(0)
    slot = jax.lax.rem(c, _NBUF)
    look = _NBUF - 1

    @pl.when(c == 0)
    def _():
        for d in range(look):
            _start_copy(adj_hbm, bufs, sems, d, d)
        x_copy = pltpu.make_async_copy(x_hbm, x_ref, x_sem)
        x_copy.start()
        x_copy.wait()
        sup_ref[...] = jnp.dot(
            x_ref[...].astype(jnp.bfloat16),
            w_ref[...].astype(jnp.bfloat16),
            preferred_element_type=jnp.float32).astype(jnp.bfloat16)

    @pl.when(c + look < nc)
    def _():
        _start_copy(adj_hbm, bufs, sems, c + look,
                    jax.lax.rem(c + look, _NBUF))

    pltpu.make_async_copy(
        adj_hbm.at[pl.ds(c * _CHUNK, _CHUNK), :],
        bufs.at[slot],
        sems.at[slot],
    ).wait()

    o_ref[...] = jnp.dot(bufs[slot].astype(jnp.bfloat16), sup_ref[...],
                         preferred_element_type=jnp.float32)


@jax.jit
def kernel(input, adj, weight):
    n, d_in = input.shape
    d_out = weight.shape[1]

    out = pl.pallas_call(
        _fused_body,
        grid=(n // _CHUNK,),
        in_specs=[
            pl.BlockSpec(memory_space=pl.ANY),
            pl.BlockSpec(memory_space=pl.ANY),
            pl.BlockSpec((d_in, d_out), lambda i: (0, 0)),
        ],
        out_specs=pl.BlockSpec((_CHUNK, d_out), lambda i: (i, 0)),
        out_shape=jax.ShapeDtypeStruct((n, d_out), jnp.float32),
        scratch_shapes=[
            pltpu.VMEM((n, d_out), jnp.bfloat16),
            pltpu.VMEM((_NBUF, _CHUNK, n), jnp.float32),
            pltpu.VMEM((n, d_in), jnp.float32),
            pltpu.SemaphoreType.DMA,
            pltpu.SemaphoreType.DMA((_NBUF,)),
        ],
        compiler_params=pltpu.CompilerParams(
            dimension_semantics=("arbitrary",)),
    )(adj, input, weight)
    return out


# speedup vs baseline: 1.0287x; 1.0287x over previous
import jax
import jax.numpy as jnp
from jax.experimental import pallas as pl
from jax.experimental.pallas import tpu as pltpu

_NSPLIT = 2
_BM_SUB = 200


def _fused_body(*refs):
    adj_refs = refs[:_NSPLIT]
    x_ref, w_ref, o_ref, sup_ref = refs[_NSPLIT:]

    @pl.when(pl.program_id(0) == 0)
    def _():
        sup_ref[...] = jnp.dot(
            x_ref[...].astype(jnp.bfloat16),
            w_ref[...].astype(jnp.bfloat16),
            preferred_element_type=jnp.float32).astype(jnp.bfloat16)

    for j, a_ref in enumerate(adj_refs):
        o_ref[j * _BM_SUB:(j + 1) * _BM_SUB, :] = jnp.dot(
            a_ref[...].astype(jnp.bfloat16), sup_ref[...],
            preferred_element_type=jnp.float32)


@jax.jit
def kernel(input, adj, weight):
    n, d_in = input.shape
    d_out = weight.shape[1]

    bm = _NSPLIT * _BM_SUB
    adj_specs = [
        pl.BlockSpec((_BM_SUB, n), lambda i, j=j: (i * _NSPLIT + j, 0))
        for j in range(_NSPLIT)
    ]
    out = pl.pallas_call(
        _fused_body,
        grid=(n // bm,),
        in_specs=adj_specs + [
            pl.BlockSpec((n, d_in), lambda i: (0, 0)),
            pl.BlockSpec((d_in, d_out), lambda i: (0, 0)),
        ],
        out_specs=pl.BlockSpec((bm, d_out), lambda i: (i, 0)),
        out_shape=jax.ShapeDtypeStruct((n, d_out), jnp.float32),
        scratch_shapes=[pltpu.VMEM((n, d_out), jnp.bfloat16)],
        compiler_params=pltpu.CompilerParams(
            dimension_semantics=("arbitrary",)),
    )(*([adj] * _NSPLIT), input, weight)
    return out
